# baseline (speedup 1.0000x reference)
import functools

import jax
import jax.numpy as jnp
from jax import lax
from jax.experimental import pallas as pl
from jax.experimental.pallas import tpu as pltpu
from jax.experimental.pallas import tpu_sc as plsc

HIDDEN = 128
E = 320000
VOCAB = 100
NUM_CORES = 2
NUM_SUBCORES = 16
NUM_WORKERS = NUM_CORES * NUM_SUBCORES
PER_WORKER = E // NUM_WORKERS
CHUNK = 128
NUM_FULL = PER_WORKER // CHUNK
TAIL = PER_WORKER - NUM_FULL * CHUNK
LANES = 16

_mesh = plsc.VectorSubcoreMesh(core_axis_name="c", subcore_axis_name="s")


@functools.partial(
    pl.kernel,
    mesh=_mesh,
    out_type=jax.ShapeDtypeStruct((E, HIDDEN), jnp.float32),
    scratch_types=[
        pltpu.VMEM((PER_WORKER,), jnp.int32),
        pltpu.VMEM((PER_WORKER,), jnp.int32),
        pltpu.VMEM((PER_WORKER,), jnp.int32),
        pltpu.VMEM((CHUNK,), jnp.int32),
        pltpu.VMEM((TAIL,), jnp.int32),
        pltpu.VMEM((CHUNK, HIDDEN), jnp.float32),
        pltpu.VMEM((CHUNK, HIDDEN), jnp.float32),
        pltpu.VMEM((CHUNK, HIDDEN), jnp.float32),
        pltpu.VMEM_SHARED((3 * VOCAB, HIDDEN), jnp.float32),
        pltpu.VMEM_SHARED((NUM_SUBCORES * CHUNK, HIDDEN), jnp.float32),
        pltpu.SemaphoreType.DMA,
        pltpu.SemaphoreType.DMA,
        pltpu.SemaphoreType.DMA,
        pltpu.SemaphoreType.DMA,
    ],
)
def _bond_encoder_sc(i0_hbm, i1_hbm, i2_hbm, t0_hbm, t1_hbm, t2_hbm,
                     out_hbm, i0_v, i1_v, i2_v, ids_v, idst_v,
                     b0_v, b1_v, b2_v,
                     tab_sh, acc_sh, g0, g1, g2, w0):
    sid = lax.axis_index("s")
    wid = sid * NUM_CORES + lax.axis_index("c")
    base = wid * PER_WORKER

    bufs = (b0_v, b1_v, b2_v)
    gsems = (g0, g1, g2)
    idx_v = (i0_v, i1_v, i2_v)

    @pl.when(sid == 0)
    def _stage():
        pltpu.sync_copy(t0_hbm, tab_sh.at[pl.ds(0, VOCAB)])
        pltpu.sync_copy(t1_hbm, tab_sh.at[pl.ds(VOCAB, VOCAB)])
        pltpu.sync_copy(t2_hbm, tab_sh.at[pl.ds(2 * VOCAB, VOCAB)])

    pltpu.sync_copy(i0_hbm.at[pl.ds(base, PER_WORKER)], i0_v)
    pltpu.sync_copy(i1_hbm.at[pl.ds(base, PER_WORKER)], i1_v)
    pltpu.sync_copy(i2_hbm.at[pl.ds(base, PER_WORKER)], i2_v)

    def rebase(j, carry):
        sl = pl.ds(j * LANES, LANES)
        i1_v[sl] = i1_v[sl] + VOCAB
        i2_v[sl] = i2_v[sl] + 2 * VOCAB
        return carry

    lax.fori_loop(0, PER_WORKER // LANES, rebase, 0)

    def build_ids(j, carry):
        sl = pl.ds(j * LANES, LANES)
        ids_v[sl] = lax.iota(jnp.int32, LANES) + (sid * CHUNK + j * LANES)
        return carry

    lax.fori_loop(0, CHUNK // LANES, build_ids, 0)
    idst_v[pl.ds(0, LANES)] = lax.iota(jnp.int32, LANES) + sid * CHUNK

    plsc.subcore_barrier()

    def gather_descr(c, t, n=CHUNK):
        sl = pl.ds(c * CHUNK, n)
        return pltpu.make_async_copy(
            tab_sh.at[idx_v[t].at[sl]], bufs[t].at[pl.ds(0, n)], gsems[t])

    def process(c, n=CHUNK, ids=None):
        reg = acc_sh.at[pl.ds(sid * CHUNK, n)]
        if ids is None:
            ids = ids_v
        for t in range(3):
            gather_descr(c, t, n).start()
        gather_descr(c, 0, n).wait()
        pltpu.sync_copy(b0_v.at[pl.ds(0, n)], reg)
        gather_descr(c, 1, n).wait()
        pltpu.sync_copy(b1_v.at[pl.ds(0, n)], acc_sh.at[ids], add=True)
        gather_descr(c, 2, n).wait()
        pltpu.sync_copy(b2_v.at[pl.ds(0, n)], acc_sh.at[ids], add=True)
        wo = pltpu.make_async_copy(
            reg, out_hbm.at[pl.ds(base + c * CHUNK, n)], w0)
        wo.start()
        wo.wait()

    def body(c, carry):
        process(c)
        return carry

    lax.fori_loop(0, NUM_FULL, body, 0)
    process(NUM_FULL, TAIL, idst_v)


def kernel(edge_attr, emb0, emb1, emb2):
    a = edge_attr.astype(jnp.int32)
    i0, i1, i2 = a[:, 0], a[:, 1], a[:, 2]
    return _bond_encoder_sc(i0, i1, i2, emb0, emb1, emb2)
